# baseline (speedup 1.0000x reference)
import jax
import jax.numpy as jnp
from jax import lax
from jax.experimental import pallas as pl
from jax.experimental.pallas import tpu as pltpu
from jax.experimental.pallas import tpu_sc as plsc

BATCH = 1024
MAX_WORDS = 50
MAX_CHARS = 20
EMBED_DIM = 64
VOCAB = 1001
PK = EMBED_DIM // 2
TSTRIDE = PK + 1

NC, NS, L = 2, 16, 16
NW = NC * NS
TOTAL_WORDS = BATCH * MAX_WORDS
WPT = TOTAL_WORDS // NW
CHUNK = 400
DSTRIDE = CHUNK + 1
NCHUNK = WPT // CHUNK


def _sc_body(chars_hbm, table_hbm, out_hbm, table_v, chars_v, dm_v, out_v):
    wid = lax.axis_index("s") * NC + lax.axis_index("c")
    pltpu.sync_copy(table_hbm, table_v)
    iota = lax.iota(jnp.int32, L)
    word0 = wid * WPT
    for k in range(NCHUNK):
        cbase = word0 + k * CHUNK
        pltpu.sync_copy(
            chars_hbm.at[pl.ds(cbase * MAX_CHARS, CHUNK * MAX_CHARS)], chars_v
        )

        def group_body(g, _):
            gw = g * L
            cidx = iota * MAX_CHARS + gw * MAX_CHARS
            row0 = plsc.load_gather(chars_v, [cidx]) * TSTRIDE
            accs = tuple(
                plsc.bitcast(
                    plsc.load_gather(table_v, [row0 + j]), jnp.bfloat16
                )
                for j in range(PK)
            )

            def c_body(c, accs):
                row = plsc.load_gather(chars_v, [cidx + c]) * TSTRIDE
                return tuple(
                    jnp.maximum(
                        accs[j],
                        plsc.bitcast(
                            plsc.load_gather(table_v, [row + j]), jnp.bfloat16
                        ),
                    )
                    for j in range(PK)
                )

            accs = lax.fori_loop(1, MAX_CHARS, c_body, accs)
            for j in range(PK):
                lo, hi = plsc.unpack(accs[j], format=plsc.PackFormat.INTERLEAVED)
                dm_v[pl.ds((2 * j) * DSTRIDE + gw, L)] = lo
                dm_v[pl.ds((2 * j + 1) * DSTRIDE + gw, L)] = hi
            return 0

        lax.fori_loop(0, CHUNK // L, group_body, 0)

        def comp_body(w, _):
            for j2 in range(EMBED_DIM // L):
                v = plsc.load_gather(dm_v, [(j2 * L + iota) * DSTRIDE + w])
                out_v[pl.ds(w * EMBED_DIM + j2 * L, L)] = v
            return 0

        lax.fori_loop(0, CHUNK, comp_body, 0, unroll=4)
        pltpu.sync_copy(
            out_v, out_hbm.at[pl.ds(cbase * EMBED_DIM, CHUNK * EMBED_DIM)]
        )


def kernel(words, chars, table):
    del words
    chars_flat = chars.reshape(-1)
    if chars_flat.dtype != jnp.int32:
        chars_flat = chars_flat.astype(jnp.int32)
    table_bf = table.astype(jnp.bfloat16).reshape(VOCAB, PK, 2)
    table_pk = lax.bitcast_convert_type(table_bf, jnp.int32)
    table_pk = jnp.pad(table_pk, ((0, 0), (0, TSTRIDE - PK))).reshape(-1)
    mesh = plsc.VectorSubcoreMesh(core_axis_name="c", subcore_axis_name="s")
    run = pl.kernel(
        _sc_body,
        out_type=jax.ShapeDtypeStruct((TOTAL_WORDS * EMBED_DIM,), jnp.float32),
        mesh=mesh,
        scratch_types=[
            pltpu.VMEM((VOCAB * TSTRIDE,), jnp.int32),
            pltpu.VMEM((CHUNK * MAX_CHARS,), jnp.int32),
            pltpu.VMEM((EMBED_DIM * DSTRIDE,), jnp.float32),
            pltpu.VMEM((CHUNK * EMBED_DIM,), jnp.float32),
        ],
        compiler_params=pltpu.CompilerParams(needs_layout_passes=False),
    )
    out = run(chars_flat, table_pk)
    return out.reshape(BATCH, MAX_WORDS, EMBED_DIM)


# speedup vs baseline: 1.0344x; 1.0344x over previous
import jax
import jax.numpy as jnp
from jax import lax
from jax.experimental import pallas as pl
from jax.experimental.pallas import tpu as pltpu
from jax.experimental.pallas import tpu_sc as plsc

BATCH = 1024
MAX_WORDS = 50
MAX_CHARS = 20
EMBED_DIM = 64
VOCAB = 1001
PK = EMBED_DIM // 2
TSTRIDE = PK + 1

NC, NS, L = 2, 16, 16
NW = NC * NS
TOTAL_WORDS = BATCH * MAX_WORDS
WPT = TOTAL_WORDS // NW
CHUNK = 400
DSTRIDE = CHUNK + 1
NCHUNK = WPT // CHUNK


def _sc_body(
    chars_hbm, table_hbm, out_hbm,
    table_v, c0_v, c1_v, dm_v, o0_v, o1_v,
    sem_t, sem_c0, sem_c1, sem_o0, sem_o1,
):
    wid = lax.axis_index("s") * NC + lax.axis_index("c")
    iota = lax.iota(jnp.int32, L)
    word0 = wid * WPT
    cbufs, csems = [c0_v, c1_v], [sem_c0, sem_c1]
    obufs, osems = [o0_v, o1_v], [sem_o0, sem_o1]

    def chars_start(k):
        cb = word0 + k * CHUNK
        return pltpu.async_copy(
            chars_hbm.at[pl.ds(cb * MAX_CHARS, CHUNK * MAX_CHARS)],
            cbufs[k % 2],
            csems[k % 2],
        )

    tdesc = pltpu.async_copy(table_hbm, table_v, sem_t)
    cdesc = [chars_start(0), None]
    odesc = [None, None]
    tdesc.wait()
    for k in range(NCHUNK):
        cbase = word0 + k * CHUNK
        if k + 1 < NCHUNK:
            cdesc[(k + 1) % 2] = chars_start(k + 1)
        cdesc[k % 2].wait()
        chars_v = cbufs[k % 2]
        out_v = obufs[k % 2]

        def group_body(g, _):
            gw = g * L
            cidx = iota * MAX_CHARS + gw * MAX_CHARS
            row0 = plsc.load_gather(chars_v, [cidx]) * TSTRIDE
            accs = tuple(
                plsc.bitcast(
                    plsc.load_gather(table_v, [row0 + j]), jnp.bfloat16
                )
                for j in range(PK)
            )

            def c_body(c, accs):
                row = plsc.load_gather(chars_v, [cidx + c]) * TSTRIDE
                return tuple(
                    jnp.maximum(
                        accs[j],
                        plsc.bitcast(
                            plsc.load_gather(table_v, [row + j]), jnp.bfloat16
                        ),
                    )
                    for j in range(PK)
                )

            accs = lax.fori_loop(1, MAX_CHARS, c_body, accs)
            for j in range(PK):
                lo, hi = plsc.unpack(accs[j], format=plsc.PackFormat.INTERLEAVED)
                dm_v[pl.ds((2 * j) * DSTRIDE + gw, L)] = lo
                dm_v[pl.ds((2 * j + 1) * DSTRIDE + gw, L)] = hi
            return 0

        lax.fori_loop(0, CHUNK // L, group_body, 0)

        def comp_body(w, _):
            for j2 in range(EMBED_DIM // L):
                v = plsc.load_gather(dm_v, [(j2 * L + iota) * DSTRIDE + w])
                out_v[pl.ds(w * EMBED_DIM + j2 * L, L)] = v
            return 0

        if k >= 2:
            odesc[k % 2].wait()
        lax.fori_loop(0, CHUNK, comp_body, 0, unroll=4)
        odesc[k % 2] = pltpu.async_copy(
            out_v,
            out_hbm.at[pl.ds(cbase * EMBED_DIM, CHUNK * EMBED_DIM)],
            osems[k % 2],
        )
    odesc[(NCHUNK - 2) % 2].wait()
    odesc[(NCHUNK - 1) % 2].wait()


def kernel(words, chars, table):
    del words
    chars_flat = chars.reshape(-1)
    if chars_flat.dtype != jnp.int32:
        chars_flat = chars_flat.astype(jnp.int32)
    table_bf = table.astype(jnp.bfloat16).reshape(VOCAB, PK, 2)
    table_pk = lax.bitcast_convert_type(table_bf, jnp.int32)
    table_pk = jnp.pad(table_pk, ((0, 0), (0, TSTRIDE - PK))).reshape(-1)
    mesh = plsc.VectorSubcoreMesh(core_axis_name="c", subcore_axis_name="s")
    run = pl.kernel(
        _sc_body,
        out_type=jax.ShapeDtypeStruct((TOTAL_WORDS * EMBED_DIM,), jnp.float32),
        mesh=mesh,
        scratch_types=[
            pltpu.VMEM((VOCAB * TSTRIDE,), jnp.int32),
            pltpu.VMEM((CHUNK * MAX_CHARS,), jnp.int32),
            pltpu.VMEM((CHUNK * MAX_CHARS,), jnp.int32),
            pltpu.VMEM((EMBED_DIM * DSTRIDE,), jnp.float32),
            pltpu.VMEM((CHUNK * EMBED_DIM,), jnp.float32),
            pltpu.VMEM((CHUNK * EMBED_DIM,), jnp.float32),
            pltpu.SemaphoreType.DMA,
            pltpu.SemaphoreType.DMA,
            pltpu.SemaphoreType.DMA,
            pltpu.SemaphoreType.DMA,
            pltpu.SemaphoreType.DMA,
        ],
        compiler_params=pltpu.CompilerParams(needs_layout_passes=False),
    )
    out = run(chars_flat, table_pk)
    return out.reshape(BATCH, MAX_WORDS, EMBED_DIM)
